# initial kernel scaffold (unmeasured)
import jax
import jax.numpy as jnp
from jax import lax
from jax.experimental import pallas as pl
from jax.experimental.pallas import tpu as pltpu

N_DEV = 32
SEQ_GLOBAL = 4096
WINDOW = 128


def kernel(x, Wq, K_ext, V_ext, Wo):
    B, S, Dm = x.shape
    _, _, Hq, Dh = K_ext.shape
    Do = Wo.shape[1]

    def body(x_ref, wq_ref, k_ref, v_ref, wo_ref, out_ref,
             kl_ref, kr_ref, vl_ref, vr_ref, send_sems, recv_sems):
        me = lax.axis_index("i")
        left = lax.rem(me - 1 + N_DEV, N_DEV)
        right = lax.rem(me + 1, N_DEV)

        barrier = pltpu.get_barrier_semaphore()
        for nbr in (left, right):
            pl.semaphore_signal(barrier, inc=1, device_id=(nbr,),
                                device_id_type=pl.DeviceIdType.MESH)
        pl.semaphore_wait(barrier, 2)

        rdmas = []
        for i, (src, dst, tgt) in enumerate((
            (k_ref, kl_ref, right),
            (v_ref, vl_ref, right),
            (k_ref, kr_ref, left),
            (v_ref, vr_ref, left),
        )):
            r = pltpu.make_async_remote_copy(
                src_ref=src, dst_ref=dst,
                send_sem=send_sems.at[i], recv_sem=recv_sems.at[i],
                device_id=(tgt,), device_id_type=pl.DeviceIdType.MESH)
            r.start()
            rdmas.append(r)

        xb = x_ref[...].astype(jnp.bfloat16).reshape(B * S, Dm)
        q = jnp.dot(xb, wq_ref[...].astype(jnp.bfloat16),
                    preferred_element_type=jnp.float32)
        q = q.reshape(B, S, Hq, Dh).astype(jnp.bfloat16)

        qi = me * S + lax.broadcasted_iota(jnp.int32, (S, 3 * S), 0)
        kj = me * S - S + lax.broadcasted_iota(jnp.int32, (S, 3 * S), 1)
        mask = (jnp.abs(qi - kj) <= WINDOW) & (kj >= 0) & (kj < SEQ_GLOBAL)

        for r in rdmas:
            r.wait()

        k_full = jnp.concatenate(
            [kl_ref[...], k_ref[...], kr_ref[...]], axis=1
        ).astype(jnp.bfloat16)
        v_full = jnp.concatenate(
            [vl_ref[...], v_ref[...], vr_ref[...]], axis=1
        ).astype(jnp.bfloat16)

        wo = wo_ref[...].astype(jnp.bfloat16)
        for b in range(B):
            ctx_heads = []
            for h in range(Hq):
                qbh = q[b, :, h, :]
                kbh = k_full[b, :, h, :]
                vbh = v_full[b, :, h, :]
                s = lax.dot_general(
                    qbh, kbh, (((1,), (1,)), ((), ())),
                    preferred_element_type=jnp.float32) * 0.125
                s = jnp.where(mask, s, -1e9)
                m = jnp.max(s, axis=-1, keepdims=True)
                w = jnp.exp(s - m)
                w = w / jnp.sum(w, axis=-1, keepdims=True)
                ctx = jnp.dot(w.astype(jnp.bfloat16), vbh,
                              preferred_element_type=jnp.float32)
                ctx_heads.append(ctx.astype(jnp.bfloat16))
            ctx_b = jnp.concatenate(ctx_heads, axis=-1)
            out_ref[b, :, :] = jnp.dot(ctx_b, wo,
                                       preferred_element_type=jnp.float32)

    return pl.pallas_call(
        body,
        out_shape=jax.ShapeDtypeStruct((B, S, Do), jnp.float32),
        in_specs=[pl.BlockSpec(memory_space=pltpu.VMEM)] * 5,
        out_specs=pl.BlockSpec(memory_space=pltpu.VMEM),
        scratch_shapes=[
            pltpu.VMEM((B, S, Hq, Dh), K_ext.dtype),
            pltpu.VMEM((B, S, Hq, Dh), K_ext.dtype),
            pltpu.VMEM((B, S, Hq, Dh), V_ext.dtype),
            pltpu.VMEM((B, S, Hq, Dh), V_ext.dtype),
            pltpu.SemaphoreType.DMA((4,)),
            pltpu.SemaphoreType.DMA((4,)),
        ],
        compiler_params=pltpu.CompilerParams(collective_id=0),
    )(x, Wq, K_ext, V_ext, Wo)


# baseline (device time: 33588 ns/iter reference)
import jax
import jax.numpy as jnp
from jax import lax
from jax.experimental import pallas as pl
from jax.experimental.pallas import tpu as pltpu

N_DEV = 32
SEQ_GLOBAL = 4096
WINDOW = 128


def kernel(x, Wq, K_ext, V_ext, Wo):
    B, S, Dm = x.shape
    _, _, Hq, Dh = K_ext.shape
    Do = Wo.shape[1]

    def body(x_ref, wq_ref, k_ref, v_ref, wo_ref, out_ref,
             kl_ref, kr_ref, vl_ref, vr_ref, send_sems, recv_sems):
        me = lax.axis_index("i")
        left = lax.rem(me - 1 + N_DEV, N_DEV)
        right = lax.rem(me + 1, N_DEV)

        barrier = pltpu.get_barrier_semaphore()
        for nbr in (left, right):
            pl.semaphore_signal(barrier, inc=1, device_id=(nbr,),
                                device_id_type=pl.DeviceIdType.MESH)
        pl.semaphore_wait(barrier, 2)

        rdmas = []
        for i, (src, dst, tgt) in enumerate((
            (k_ref, kl_ref, right),
            (v_ref, vl_ref, right),
            (k_ref, kr_ref, left),
            (v_ref, vr_ref, left),
        )):
            r = pltpu.make_async_remote_copy(
                src_ref=src, dst_ref=dst,
                send_sem=send_sems.at[i], recv_sem=recv_sems.at[i],
                device_id=(tgt,), device_id_type=pl.DeviceIdType.MESH)
            r.start()
            rdmas.append(r)

        xb = x_ref[...].reshape(B * S, Dm)
        q = jnp.dot(xb, wq_ref[...], preferred_element_type=jnp.float32)
        q = q.reshape(B, S, Hq, Dh)

        qi = me * S + lax.broadcasted_iota(jnp.int32, (S, 3 * S), 0)
        kj = me * S - S + lax.broadcasted_iota(jnp.int32, (S, 3 * S), 1)
        mask = (jnp.abs(qi - kj) <= WINDOW) & (kj >= 0) & (kj < SEQ_GLOBAL)

        for r in rdmas:
            r.wait()

        k_full = jnp.concatenate(
            [kl_ref[...], k_ref[...], kr_ref[...]], axis=1
        )
        v_full = jnp.concatenate(
            [vl_ref[...], v_ref[...], vr_ref[...]], axis=1
        )

        wo = wo_ref[...]
        for b in range(B):
            ctx_heads = []
            for h in range(Hq):
                qbh = q[b, :, h, :]
                kbh = k_full[b, :, h, :]
                vbh = v_full[b, :, h, :]
                s = lax.dot_general(
                    qbh, kbh, (((1,), (1,)), ((), ())),
                    preferred_element_type=jnp.float32) * 0.125
                s = jnp.where(mask, s, -1e9)
                m = jnp.max(s, axis=-1, keepdims=True)
                w = jnp.exp(s - m)
                w = w / jnp.sum(w, axis=-1, keepdims=True)
                ctx = jnp.dot(w, vbh,
                              preferred_element_type=jnp.float32)
                ctx_heads.append(ctx)
            ctx_b = jnp.concatenate(ctx_heads, axis=-1)
            out_ref[b, :, :] = jnp.dot(ctx_b, wo,
                                       preferred_element_type=jnp.float32)

    return pl.pallas_call(
        body,
        out_shape=jax.ShapeDtypeStruct((B, S, Do), jnp.float32),
        in_specs=[pl.BlockSpec(memory_space=pltpu.VMEM)] * 5,
        out_specs=pl.BlockSpec(memory_space=pltpu.VMEM),
        scratch_shapes=[
            pltpu.VMEM((B, S, Hq, Dh), K_ext.dtype),
            pltpu.VMEM((B, S, Hq, Dh), K_ext.dtype),
            pltpu.VMEM((B, S, Hq, Dh), V_ext.dtype),
            pltpu.VMEM((B, S, Hq, Dh), V_ext.dtype),
            pltpu.SemaphoreType.DMA((4,)),
            pltpu.SemaphoreType.DMA((4,)),
        ],
        compiler_params=pltpu.CompilerParams(collective_id=0),
    )(x, Wq, K_ext, V_ext, Wo)


# device time: 23356 ns/iter; 1.4381x vs baseline; 1.4381x over previous
import jax
import jax.numpy as jnp
from jax import lax
from jax.experimental import pallas as pl
from jax.experimental.pallas import tpu as pltpu

N_DEV = 32
SEQ_GLOBAL = 4096
WINDOW = 128


def kernel(x, Wq, K_ext, V_ext, Wo):
    B, S, Dm = x.shape
    _, _, Hq, Dh = K_ext.shape
    Do = Wo.shape[1]

    def body(x_ref, wq_ref, k_ref, v_ref, wo_ref, out_ref,
             kv_send, kv_l, kv_r, send_sems, recv_sems):
        me = lax.axis_index("i")
        left = lax.rem(me - 1 + N_DEV, N_DEV)
        right = lax.rem(me + 1, N_DEV)

        barrier = pltpu.get_barrier_semaphore()
        for nbr in (left, right):
            pl.semaphore_signal(barrier, inc=1, device_id=(nbr,),
                                device_id_type=pl.DeviceIdType.MESH)
        kv_send[0] = k_ref[...].astype(jnp.bfloat16)
        kv_send[1] = v_ref[...].astype(jnp.bfloat16)
        pl.semaphore_wait(barrier, 2)

        rdmas = []
        for i, (dst, tgt) in enumerate(((kv_l, right), (kv_r, left))):
            r = pltpu.make_async_remote_copy(
                src_ref=kv_send, dst_ref=dst,
                send_sem=send_sems.at[i], recv_sem=recv_sems.at[i],
                device_id=(tgt,), device_id_type=pl.DeviceIdType.MESH)
            r.start()
            rdmas.append(r)

        xb = x_ref[...].reshape(B * S, Dm)
        q = jnp.dot(xb, wq_ref[...], preferred_element_type=jnp.float32)
        q = q.reshape(B, S, Hq, Dh)

        qi = me * S + lax.broadcasted_iota(jnp.int32, (S, 3 * S), 0)
        kj = me * S - S + lax.broadcasted_iota(jnp.int32, (S, 3 * S), 1)
        mask = (jnp.abs(qi - kj) <= WINDOW) & (kj >= 0) & (kj < SEQ_GLOBAL)

        for r in rdmas:
            r.wait_recv()

        k_full = jnp.concatenate(
            [kv_l[0].astype(jnp.float32), k_ref[...],
             kv_r[0].astype(jnp.float32)], axis=1)
        v_full = jnp.concatenate(
            [kv_l[1].astype(jnp.float32), v_ref[...],
             kv_r[1].astype(jnp.float32)], axis=1)

        wo = wo_ref[...]
        for b in range(B):
            ctx_heads = []
            for h in range(Hq):
                qbh = q[b, :, h, :]
                kbh = k_full[b, :, h, :]
                vbh = v_full[b, :, h, :]
                s = lax.dot_general(
                    qbh, kbh, (((1,), (1,)), ((), ())),
                    preferred_element_type=jnp.float32) * 0.125
                s = jnp.where(mask, s, -1e9)
                m = jnp.max(s, axis=-1, keepdims=True)
                w = jnp.exp(s - m)
                w = w / jnp.sum(w, axis=-1, keepdims=True)
                ctx = jnp.dot(w, vbh,
                              preferred_element_type=jnp.float32)
                ctx_heads.append(ctx)
            ctx_b = jnp.concatenate(ctx_heads, axis=-1)
            out_ref[b, :, :] = jnp.dot(ctx_b, wo,
                                       preferred_element_type=jnp.float32)

        for r in rdmas:
            r.wait_send()

    return pl.pallas_call(
        body,
        out_shape=jax.ShapeDtypeStruct((B, S, Do), jnp.float32),
        in_specs=[pl.BlockSpec(memory_space=pltpu.VMEM)] * 5,
        out_specs=pl.BlockSpec(memory_space=pltpu.VMEM),
        scratch_shapes=[
            pltpu.VMEM((2, B, S, Hq, Dh), jnp.bfloat16),
            pltpu.VMEM((2, B, S, Hq, Dh), jnp.bfloat16),
            pltpu.VMEM((2, B, S, Hq, Dh), jnp.bfloat16),
            pltpu.SemaphoreType.DMA((2,)),
            pltpu.SemaphoreType.DMA((2,)),
        ],
        compiler_params=pltpu.CompilerParams(collective_id=0),
    )(x, Wq, K_ext, V_ext, Wo)


# device time: 21682 ns/iter; 1.5491x vs baseline; 1.0772x over previous
import jax
import jax.numpy as jnp
from jax import lax
from jax.experimental import pallas as pl
from jax.experimental.pallas import tpu as pltpu

N_DEV = 32
SEQ_GLOBAL = 4096
WINDOW = 128


def kernel(x, Wq, K_ext, V_ext, Wo):
    B, S, Dm = x.shape
    _, _, Hq, Dh = K_ext.shape
    Do = Wo.shape[1]

    def body(x_ref, wq_ref, k_ref, v_ref, wo_ref, out_ref,
             kv_send, kv_l, kv_r, send_sems, recv_sems):
        me = lax.axis_index("i")
        left = lax.rem(me - 1 + N_DEV, N_DEV)
        right = lax.rem(me + 1, N_DEV)

        kv_send[0] = k_ref[...].astype(jnp.bfloat16)
        kv_send[1] = v_ref[...].astype(jnp.bfloat16)

        barrier = pltpu.get_barrier_semaphore()
        for nbr in (left, right):
            pl.semaphore_signal(barrier, inc=1, device_id=(nbr,),
                                device_id_type=pl.DeviceIdType.MESH)
        pl.semaphore_wait(barrier, 2)

        rdmas = []
        for i, (dst, tgt) in enumerate(((kv_l, right), (kv_r, left))):
            r = pltpu.make_async_remote_copy(
                src_ref=kv_send, dst_ref=dst,
                send_sem=send_sems.at[i], recv_sem=recv_sems.at[i],
                device_id=(tgt,), device_id_type=pl.DeviceIdType.MESH)
            r.start()
            rdmas.append(r)

        xb = x_ref[...].reshape(B * S, Dm)
        q = jnp.dot(xb, wq_ref[...], preferred_element_type=jnp.float32)
        q = q.reshape(B, S, Hq, Dh)

        k_own = k_ref[...]
        v_own = v_ref[...]
        m1s, l1s, accs = [], [], []
        for b in range(B):
            for h in range(Hq):
                qbh = q[b, :, h, :]
                s_o = lax.dot_general(
                    qbh, k_own[b, :, h, :], (((1,), (1,)), ((), ())),
                    preferred_element_type=jnp.float32) * 0.125
                m1 = jnp.max(s_o, axis=-1, keepdims=True)
                p_o = jnp.exp(s_o - m1)
                l1 = jnp.sum(p_o, axis=-1, keepdims=True)
                acc = jnp.dot(p_o, v_own[b, :, h, :],
                              preferred_element_type=jnp.float32)
                m1s.append(m1)
                l1s.append(l1)
                accs.append(acc)

        has_l = me != 0
        has_r = me != N_DEV - 1
        ii = lax.broadcasted_iota(jnp.int32, (S, S), 0)
        jj = lax.broadcasted_iota(jnp.int32, (S, S), 1)
        mask_h = jnp.concatenate(
            [(ii <= jj) & has_l, (ii >= jj) & has_r], axis=1)

        for r in rdmas:
            r.wait()

        k_halo = jnp.concatenate(
            [kv_l[0], kv_r[0]], axis=1).astype(jnp.float32)
        v_halo = jnp.concatenate(
            [kv_l[1], kv_r[1]], axis=1).astype(jnp.float32)

        wo = wo_ref[...]
        idx = 0
        for b in range(B):
            ctx_heads = []
            for h in range(Hq):
                qbh = q[b, :, h, :]
                s_h = lax.dot_general(
                    qbh, k_halo[b, :, h, :], (((1,), (1,)), ((), ())),
                    preferred_element_type=jnp.float32) * 0.125
                s_h = jnp.where(mask_h, s_h, -1e9)
                m1, l1, acc1 = m1s[idx], l1s[idx], accs[idx]
                idx += 1
                m2 = jnp.maximum(m1, jnp.max(s_h, axis=-1, keepdims=True))
                p_h = jnp.exp(s_h - m2)
                c = jnp.exp(m1 - m2)
                acc = acc1 * c + jnp.dot(p_h, v_halo[b, :, h, :],
                                         preferred_element_type=jnp.float32)
                l = l1 * c + jnp.sum(p_h, axis=-1, keepdims=True)
                ctx_heads.append(acc / l)
            ctx_b = jnp.concatenate(ctx_heads, axis=-1)
            out_ref[b, :, :] = jnp.dot(ctx_b, wo,
                                       preferred_element_type=jnp.float32)

    return pl.pallas_call(
        body,
        out_shape=jax.ShapeDtypeStruct((B, S, Do), jnp.float32),
        in_specs=[pl.BlockSpec(memory_space=pltpu.VMEM)] * 5,
        out_specs=pl.BlockSpec(memory_space=pltpu.VMEM),
        scratch_shapes=[
            pltpu.VMEM((2, B, S, Hq, Dh), jnp.bfloat16),
            pltpu.VMEM((2, B, S, Hq, Dh), jnp.bfloat16),
            pltpu.VMEM((2, B, S, Hq, Dh), jnp.bfloat16),
            pltpu.SemaphoreType.DMA((2,)),
            pltpu.SemaphoreType.DMA((2,)),
        ],
        compiler_params=pltpu.CompilerParams(collective_id=0),
    )(x, Wq, K_ext, V_ext, Wo)
